# baseline (speedup 1.0000x reference)
import functools

import jax
import jax.numpy as jnp
from jax import lax
from jax.experimental import pallas as pl
from jax.experimental.pallas import tpu as pltpu
from jax.experimental.pallas import tpu_sc as plsc

_V, _D = 1_000_000, 32
_B = 1_048_576
_DP = 128

_NC, _NS = 2, 16
_NW = _NC * _NS
_L = 16

_SLAB = 512
_NSLAB = 999_936 // _SLAB
_TAIL = _V - _NSLAB * _SLAB


def _transpose_block(src, dst, ngrp):
    iot = lax.iota(jnp.int32, _L)
    c1 = iot + _L

    def row(i, carry):
        ii = jnp.full((_L,), i, jnp.int32)
        x0 = plsc.load_gather(src, [iot, ii])
        x1 = plsc.load_gather(src, [c1, ii])
        dst[i >> 2, pl.ds((i & 3) * _D, _L)] = x0
        dst[i >> 2, pl.ds((i & 3) * _D + _L, _L)] = x1
        return carry

    lax.fori_loop(0, ngrp * _L, row, 0, unroll=16)


def _relayout_body(featT_hbm, tab_hbm, sbuf, tbuf, sbuf_t, tbuf_t):
    wid = lax.axis_index("s") * _NC + lax.axis_index("c")
    nsw = jnp.where(wid == 0, (_NSLAB + _NW - 1) // _NW, _NSLAB // _NW)

    def slab(j, carry):
        r0 = pl.multiple_of((wid + _NW * j) * _SLAB, _SLAB)
        o0 = pl.multiple_of((wid + _NW * j) * (_SLAB * _D // _DP),
                            _SLAB * _D // _DP)
        pltpu.sync_copy(featT_hbm.at[:, pl.ds(r0, _SLAB)], sbuf)
        _transpose_block(sbuf, tbuf, _SLAB // _L)
        pltpu.sync_copy(tbuf, tab_hbm.at[pl.ds(o0, _SLAB * _D // _DP)])
        return carry

    lax.fori_loop(0, nsw, slab, 0)

    @pl.when(wid == 1)
    def _():
        r0 = _NSLAB * _SLAB
        pltpu.sync_copy(featT_hbm.at[:, pl.ds(r0, _TAIL)], sbuf_t)
        _transpose_block(sbuf_t, tbuf_t, _TAIL // _L)
        pltpu.sync_copy(tbuf_t, tab_hbm.at[pl.ds(r0 * _D // _DP, _TAIL * _D // _DP)])


@functools.lru_cache(maxsize=1)
def _build_relayout():
    mesh = plsc.VectorSubcoreMesh(core_axis_name="c", subcore_axis_name="s")
    return pl.kernel(
        _relayout_body,
        mesh=mesh,
        out_type=jax.ShapeDtypeStruct((_V * _D // _DP, _DP), jnp.float32),
        scratch_types=[
            pltpu.VMEM((_D, _SLAB), jnp.float32),
            pltpu.VMEM((_SLAB * _D // _DP, _DP), jnp.float32),
            pltpu.VMEM((_D, _TAIL), jnp.float32),
            pltpu.VMEM((_TAIL * _D // _DP, _DP), jnp.float32),
        ],
        compiler_params=pltpu.CompilerParams(needs_layout_passes=False),
    )


_BPW = _B // _NW
_CHUNK = 1024
_NCHUNK = _BPW // _CHUNK
_NBUF = 2


def _gather_body(table_hbm, idx_hbm, out_hbm, idx_all, rows0, rows1,
                 gs0, gs1, ws0, ws1):
    wid = lax.axis_index("s") * _NC + lax.axis_index("c")
    base = wid * _BPW

    pltpu.sync_copy(idx_hbm.at[pl.ds(base, _BPW)], idx_all)

    rows = (rows0, rows1)
    gsem = (gs0, gs1)
    wsem = (ws0, ws1)
    gd = [None] * _NCHUNK
    wd = [None] * _NCHUNK
    for c in range(_NCHUNK):
        b = c % _NBUF
        if c >= _NBUF:
            wd[c - _NBUF].wait()
        gd[c] = pltpu.async_copy(
            table_hbm.at[idx_all.at[pl.ds(c * _CHUNK, _CHUNK)]], rows[b], gsem[b])
        if c >= 1:
            bp = (c - 1) % _NBUF
            gd[c - 1].wait()
            wd[c - 1] = pltpu.async_copy(
                rows[bp],
                out_hbm.at[pl.ds(base + (c - 1) * _CHUNK, _CHUNK), pl.ds(0, _D)],
                wsem[bp])
    last = _NCHUNK - 1
    gd[last].wait()
    wd[last] = pltpu.async_copy(
        rows[last % _NBUF],
        out_hbm.at[pl.ds(base + last * _CHUNK, _CHUNK), pl.ds(0, _D)],
        wsem[last % _NBUF])
    wd[last - 1].wait()
    wd[last].wait()


@functools.lru_cache(maxsize=1)
def _build_gather():
    mesh = plsc.VectorSubcoreMesh(core_axis_name="c", subcore_axis_name="s")
    return pl.kernel(
        _gather_body,
        mesh=mesh,
        out_type=jax.ShapeDtypeStruct((_B, _DP), jnp.float32),
        scratch_types=[
            pltpu.VMEM((_BPW,), jnp.int32),
            pltpu.VMEM((_CHUNK, _D), jnp.float32),
            pltpu.VMEM((_CHUNK, _D), jnp.float32),
            pltpu.SemaphoreType.DMA,
            pltpu.SemaphoreType.DMA,
            pltpu.SemaphoreType.DMA,
            pltpu.SemaphoreType.DMA,
        ],
        compiler_params=pltpu.CompilerParams(
            use_tc_tiling_on_sc=False, needs_layout_passes=False),
    )


def kernel(features, rev):
    tab128 = _build_relayout()(features.T)
    table = tab128.reshape(_V, _D)
    out128 = _build_gather()(table, rev.astype(jnp.int32))
    return out128[:, :_D]


# speedup vs baseline: 1.2280x; 1.2280x over previous
import functools

import jax
import jax.numpy as jnp
from jax import lax
from jax.experimental import pallas as pl
from jax.experimental.pallas import tpu as pltpu
from jax.experimental.pallas import tpu_sc as plsc

_V, _D = 1_000_000, 32
_B = 1_048_576
_DP = 128

_NC, _NS = 2, 16
_NW = _NC * _NS
_L = 16

_SLAB = 512
_NSLAB = 999_936 // _SLAB
_TAIL = _V - _NSLAB * _SLAB


def _transpose_block(src, dst, ngrp):
    iot = lax.iota(jnp.int32, _L)
    rv0 = iot >> 2
    cb = (iot & 3) * _D

    @plsc.parallel_loop(0, ngrp, unroll=4)
    def _grp(g2):
        rows = rv0 + 4 * g2
        for c in range(_D):
            x = src[c, pl.ds(_L * g2, _L)]
            plsc.store_scatter(dst, [rows, cb + c], x)


def _relayout_body(featT_hbm, tab_hbm, sbuf, tbuf, sbuf_t, tbuf_t):
    wid = lax.axis_index("s") * _NC + lax.axis_index("c")
    nsw = jnp.where(wid == 0, (_NSLAB + _NW - 1) // _NW, _NSLAB // _NW)

    def slab(j, carry):
        r0 = pl.multiple_of((wid + _NW * j) * _SLAB, _SLAB)
        o0 = pl.multiple_of((wid + _NW * j) * (_SLAB * _D // _DP),
                            _SLAB * _D // _DP)
        pltpu.sync_copy(featT_hbm.at[:, pl.ds(r0, _SLAB)], sbuf)
        _transpose_block(sbuf, tbuf, _SLAB // _L)
        pltpu.sync_copy(tbuf, tab_hbm.at[pl.ds(o0, _SLAB * _D // _DP)])
        return carry

    lax.fori_loop(0, nsw, slab, 0)

    @pl.when(wid == 1)
    def _():
        r0 = _NSLAB * _SLAB
        pltpu.sync_copy(featT_hbm.at[:, pl.ds(r0, _TAIL)], sbuf_t)
        _transpose_block(sbuf_t, tbuf_t, _TAIL // _L)
        pltpu.sync_copy(tbuf_t, tab_hbm.at[pl.ds(r0 * _D // _DP, _TAIL * _D // _DP)])


@functools.lru_cache(maxsize=1)
def _build_relayout():
    mesh = plsc.VectorSubcoreMesh(core_axis_name="c", subcore_axis_name="s")
    return pl.kernel(
        _relayout_body,
        mesh=mesh,
        out_type=jax.ShapeDtypeStruct((_V * _D // _DP, _DP), jnp.float32),
        scratch_types=[
            pltpu.VMEM((_D, _SLAB), jnp.float32),
            pltpu.VMEM((_SLAB * _D // _DP, _DP), jnp.float32),
            pltpu.VMEM((_D, _TAIL), jnp.float32),
            pltpu.VMEM((_TAIL * _D // _DP, _DP), jnp.float32),
        ],
        compiler_params=pltpu.CompilerParams(needs_layout_passes=False),
    )


_BPW = _B // _NW
_CHUNK = 1024
_NCHUNK = _BPW // _CHUNK
_NBUF = 2


def _gather_body(table_hbm, idx_hbm, out_hbm, idx_all, rows0, rows1,
                 gs0, gs1, ws0, ws1):
    wid = lax.axis_index("s") * _NC + lax.axis_index("c")
    base = wid * _BPW

    pltpu.sync_copy(idx_hbm.at[pl.ds(base, _BPW)], idx_all)

    rows = (rows0, rows1)
    gsem = (gs0, gs1)
    wsem = (ws0, ws1)
    gd = [None] * _NCHUNK
    wd = [None] * _NCHUNK
    for c in range(_NCHUNK):
        b = c % _NBUF
        if c >= _NBUF:
            wd[c - _NBUF].wait()
        gd[c] = pltpu.async_copy(
            table_hbm.at[idx_all.at[pl.ds(c * _CHUNK, _CHUNK)]], rows[b], gsem[b])
        if c >= 1:
            bp = (c - 1) % _NBUF
            gd[c - 1].wait()
            wd[c - 1] = pltpu.async_copy(
                rows[bp],
                out_hbm.at[pl.ds(base + (c - 1) * _CHUNK, _CHUNK), pl.ds(0, _D)],
                wsem[bp])
    last = _NCHUNK - 1
    gd[last].wait()
    wd[last] = pltpu.async_copy(
        rows[last % _NBUF],
        out_hbm.at[pl.ds(base + last * _CHUNK, _CHUNK), pl.ds(0, _D)],
        wsem[last % _NBUF])
    wd[last - 1].wait()
    wd[last].wait()


@functools.lru_cache(maxsize=1)
def _build_gather():
    mesh = plsc.VectorSubcoreMesh(core_axis_name="c", subcore_axis_name="s")
    return pl.kernel(
        _gather_body,
        mesh=mesh,
        out_type=jax.ShapeDtypeStruct((_B, _DP), jnp.float32),
        scratch_types=[
            pltpu.VMEM((_BPW,), jnp.int32),
            pltpu.VMEM((_CHUNK, _D), jnp.float32),
            pltpu.VMEM((_CHUNK, _D), jnp.float32),
            pltpu.SemaphoreType.DMA,
            pltpu.SemaphoreType.DMA,
            pltpu.SemaphoreType.DMA,
            pltpu.SemaphoreType.DMA,
        ],
        compiler_params=pltpu.CompilerParams(
            use_tc_tiling_on_sc=False, needs_layout_passes=False),
    )


def kernel(features, rev):
    tab128 = _build_relayout()(features.T)
    table = tab128.reshape(_V, _D)
    out128 = _build_gather()(table, rev.astype(jnp.int32))
    return out128[:, :_D]


# speedup vs baseline: 1.2735x; 1.0370x over previous
import functools

import jax
import jax.numpy as jnp
from jax import lax
from jax.experimental import pallas as pl
from jax.experimental.pallas import tpu as pltpu
from jax.experimental.pallas import tpu_sc as plsc

_V, _D = 1_000_000, 32
_B = 1_048_576
_DP = 128

_NC, _NS = 2, 16
_NW = _NC * _NS
_L = 16

_SLAB = 512
_NSLAB = 999_936 // _SLAB
_TAIL = _V - _NSLAB * _SLAB


def _transpose_block(src, dst, ngrp):
    iot = lax.iota(jnp.int32, _L)
    rv0 = iot >> 2
    cb = (iot & 3) * _D

    @plsc.parallel_loop(0, ngrp * _D, unroll=8)
    def _one(k):
        g2 = k >> 5
        c = k & (_D - 1)
        x = src[c, pl.ds(_L * g2, _L)]
        plsc.store_scatter(dst, [rv0 + 4 * g2, cb + c], x)


def _relayout_body(featT_hbm, tab_hbm, sbuf, tbuf, sbuf_t, tbuf_t):
    wid = lax.axis_index("s") * _NC + lax.axis_index("c")
    nsw = jnp.where(wid == 0, (_NSLAB + _NW - 1) // _NW, _NSLAB // _NW)

    def slab(j, carry):
        r0 = pl.multiple_of((wid + _NW * j) * _SLAB, _SLAB)
        o0 = pl.multiple_of((wid + _NW * j) * (_SLAB * _D // _DP),
                            _SLAB * _D // _DP)
        pltpu.sync_copy(featT_hbm.at[:, pl.ds(r0, _SLAB)], sbuf)
        _transpose_block(sbuf, tbuf, _SLAB // _L)
        pltpu.sync_copy(tbuf, tab_hbm.at[pl.ds(o0, _SLAB * _D // _DP)])
        return carry

    lax.fori_loop(0, nsw, slab, 0)

    @pl.when(wid == 1)
    def _():
        r0 = _NSLAB * _SLAB
        pltpu.sync_copy(featT_hbm.at[:, pl.ds(r0, _TAIL)], sbuf_t)
        _transpose_block(sbuf_t, tbuf_t, _TAIL // _L)
        pltpu.sync_copy(tbuf_t, tab_hbm.at[pl.ds(r0 * _D // _DP, _TAIL * _D // _DP)])


@functools.lru_cache(maxsize=1)
def _build_relayout():
    mesh = plsc.VectorSubcoreMesh(core_axis_name="c", subcore_axis_name="s")
    return pl.kernel(
        _relayout_body,
        mesh=mesh,
        out_type=jax.ShapeDtypeStruct((_V * _D // _DP, _DP), jnp.float32),
        scratch_types=[
            pltpu.VMEM((_D, _SLAB), jnp.float32),
            pltpu.VMEM((_SLAB * _D // _DP, _DP), jnp.float32),
            pltpu.VMEM((_D, _TAIL), jnp.float32),
            pltpu.VMEM((_TAIL * _D // _DP, _DP), jnp.float32),
        ],
        compiler_params=pltpu.CompilerParams(needs_layout_passes=False),
    )


_BPW = _B // _NW
_CHUNK = 1024
_NCHUNK = _BPW // _CHUNK
_NBUF = 2


def _gather_body(table_hbm, idx_hbm, out_hbm, idx_all, rows0, rows1,
                 gs0, gs1, ws0, ws1):
    wid = lax.axis_index("s") * _NC + lax.axis_index("c")
    base = wid * _BPW

    pltpu.sync_copy(idx_hbm.at[pl.ds(base, _BPW)], idx_all)

    rows = (rows0, rows1)
    gsem = (gs0, gs1)
    wsem = (ws0, ws1)
    gd = [None] * _NCHUNK
    wd = [None] * _NCHUNK
    for c in range(_NCHUNK):
        b = c % _NBUF
        if c >= _NBUF:
            wd[c - _NBUF].wait()
        gd[c] = pltpu.async_copy(
            table_hbm.at[idx_all.at[pl.ds(c * _CHUNK, _CHUNK)]], rows[b], gsem[b])
        if c >= 1:
            bp = (c - 1) % _NBUF
            gd[c - 1].wait()
            wd[c - 1] = pltpu.async_copy(
                rows[bp],
                out_hbm.at[pl.ds(base + (c - 1) * _CHUNK, _CHUNK), pl.ds(0, _D)],
                wsem[bp])
    last = _NCHUNK - 1
    gd[last].wait()
    wd[last] = pltpu.async_copy(
        rows[last % _NBUF],
        out_hbm.at[pl.ds(base + last * _CHUNK, _CHUNK), pl.ds(0, _D)],
        wsem[last % _NBUF])
    wd[last - 1].wait()
    wd[last].wait()


@functools.lru_cache(maxsize=1)
def _build_gather():
    mesh = plsc.VectorSubcoreMesh(core_axis_name="c", subcore_axis_name="s")
    return pl.kernel(
        _gather_body,
        mesh=mesh,
        out_type=jax.ShapeDtypeStruct((_B, _DP), jnp.float32),
        scratch_types=[
            pltpu.VMEM((_BPW,), jnp.int32),
            pltpu.VMEM((_CHUNK, _D), jnp.float32),
            pltpu.VMEM((_CHUNK, _D), jnp.float32),
            pltpu.SemaphoreType.DMA,
            pltpu.SemaphoreType.DMA,
            pltpu.SemaphoreType.DMA,
            pltpu.SemaphoreType.DMA,
        ],
        compiler_params=pltpu.CompilerParams(
            use_tc_tiling_on_sc=False, needs_layout_passes=False),
    )


def kernel(features, rev):
    tab128 = _build_relayout()(features.T)
    table = tab128.reshape(_V, _D)
    out128 = _build_gather()(table, rev.astype(jnp.int32))
    return out128[:, :_D]


# speedup vs baseline: 1.4089x; 1.1063x over previous
import functools

import jax
import jax.numpy as jnp
from jax import lax
from jax.experimental import pallas as pl
from jax.experimental.pallas import tpu as pltpu
from jax.experimental.pallas import tpu_sc as plsc

_V, _D = 1_000_000, 32
_B = 1_048_576
_DP = 128

_NC, _NS = 2, 16
_NW = _NC * _NS
_BPW = _B // _NW
_CHUNK = 1024
_NCHUNK = _BPW // _CHUNK
_NBUF = 2


def _body(table_hbm, idx_hbm, out_hbm, idx_all, rows0, rows1, gs0, gs1, ws0, ws1):
    wid = lax.axis_index("s") * _NC + lax.axis_index("c")
    base = wid * _BPW

    pltpu.sync_copy(idx_hbm.at[pl.ds(base, _BPW)], idx_all)

    rows = (rows0, rows1)
    gsem = (gs0, gs1)
    wsem = (ws0, ws1)
    gd = [None] * _NCHUNK
    wd = [None] * _NCHUNK
    for c in range(_NCHUNK):
        b = c % _NBUF
        if c >= _NBUF:
            wd[c - _NBUF].wait()
        gd[c] = pltpu.async_copy(
            table_hbm.at[idx_all.at[pl.ds(c * _CHUNK, _CHUNK)]], rows[b], gsem[b])
        if c >= 1:
            bp = (c - 1) % _NBUF
            gd[c - 1].wait()
            wd[c - 1] = pltpu.async_copy(
                rows[bp],
                out_hbm.at[pl.ds(base + (c - 1) * _CHUNK, _CHUNK), pl.ds(0, _D)],
                wsem[bp])
    last = _NCHUNK - 1
    gd[last].wait()
    wd[last] = pltpu.async_copy(
        rows[last % _NBUF],
        out_hbm.at[pl.ds(base + last * _CHUNK, _CHUNK), pl.ds(0, _D)],
        wsem[last % _NBUF])
    wd[last - 1].wait()
    wd[last].wait()


@functools.lru_cache(maxsize=1)
def _build():
    mesh = plsc.VectorSubcoreMesh(core_axis_name="c", subcore_axis_name="s")
    return pl.kernel(
        _body,
        mesh=mesh,
        out_type=jax.ShapeDtypeStruct((_B, _DP), jnp.float32),
        scratch_types=[
            pltpu.VMEM((_BPW,), jnp.int32),
            pltpu.VMEM((_CHUNK, _D), jnp.float32),
            pltpu.VMEM((_CHUNK, _D), jnp.float32),
            pltpu.SemaphoreType.DMA,
            pltpu.SemaphoreType.DMA,
            pltpu.SemaphoreType.DMA,
            pltpu.SemaphoreType.DMA,
        ],
        compiler_params=pltpu.CompilerParams(
            use_tc_tiling_on_sc=False, needs_layout_passes=False),
    )


def kernel(features, rev):
    out128 = _build()(features, rev.astype(jnp.int32))
    return out128[:, :_D]
